# baseline (speedup 1.0000x reference)
import functools

import jax
import jax.numpy as jnp
from jax import lax
from jax.experimental import pallas as pl
from jax.experimental.pallas import tpu as pltpu
from jax.experimental.pallas import tpu_sc as plsc


def _score_kernel(x_ref, w1_ref, b1_ref, w2_ref, b2_ref, s_ref):
    x = x_ref[:].reshape(2048, 128)
    h = jnp.maximum(
        lax.dot_general(x, w1_ref[:], (((1,), (0,)), ((), ())),
                        preferred_element_type=jnp.float32) + b1_ref[:],
        0.0)
    s_ref[:] = lax.dot_general(w2_ref[:], h, (((0,), (1,)), ((), ())),
                               preferred_element_type=jnp.float32) + b2_ref[:]


_SEL_OUT = (
    jax.ShapeDtypeStruct((2048,), jnp.float32),
    jax.ShapeDtypeStruct((16,), jnp.int32),
    jax.ShapeDtypeStruct((16,), jnp.float32),
)


@functools.partial(
    pl.kernel,
    mesh=plsc.VectorSubcoreMesh(core_axis_name="c", subcore_axis_name="s"),
    out_type=_SEL_OUT,
    compiler_params=pltpu.CompilerParams(needs_layout_passes=False),
    scratch_types=[
        pltpu.VMEM((1024,), jnp.float32),
        pltpu.VMEM((8, 128), jnp.float32),
        pltpu.VMEM((1024,), jnp.float32),
        pltpu.VMEM((16,), jnp.int32),
        pltpu.VMEM((16,), jnp.float32),
        pltpu.SemaphoreType.DMA,
    ],
)
def _select_kernel(s_hbm, adj_hbm, sm_hbm, id_hbm, flag_hbm,
                   s_v, a_v, sm_v, id_v, flag_v, sem):
    cid = lax.axis_index("c")
    sid = lax.axis_index("s")

    for core in range(2):
        @pl.when((cid == core) & (sid == 0))
        def _(core=core):
            cps = [pltpu.async_copy(s_hbm.at[pl.ds(core * 1024, 1024)],
                                    s_v, sem)]
            for bl in range(4):
                b = core * 4 + bl
                cps.append(pltpu.async_copy(adj_hbm.at[b, pl.ds(0, 2)],
                                            a_v.at[pl.ds(2 * bl, 2)], sem))
            for cp in cps:
                cp.wait()
            iota = lax.broadcasted_iota(jnp.int32, (16,), 0)
            ids = []
            for ls in range(8):
                row = ls % 2
                v = None
                gi = None
                for c in range(8):
                    off = ls * 128 + c * 16
                    sv = s_v[pl.ds(off, 16)]
                    av = a_v[ls, pl.ds(c * 16, 16)]
                    if c == 0:
                        av = jnp.where(iota == (1 - row), 0.0, av)
                    sm = sv * av
                    sm_v[pl.ds(off, 16)] = sm
                    if c == 0:
                        v, gi = sm, iota
                    else:
                        cond = sm > v
                        gi = jnp.where(cond, c * 16 + iota, gi)
                        v = jnp.where(cond, sm, v)
                mx = jnp.max(v)
                ids.append(jnp.min(jnp.where(v == mx, gi, 2048)))
            ids_vec = jnp.zeros((16,), jnp.int32)
            for ls in range(8):
                ids_vec = jnp.where(iota == ls, ids[ls], ids_vec)
            id_v[:] = ids_vec
            flag_vec = jnp.zeros((16,), jnp.float32)
            for bl in range(4):
                a = ids[2 * bl] > 0
                o = ids[2 * bl + 1] > 0
                fb = jnp.where(a & o, 3.0,
                               jnp.where(a, 1.0, jnp.where(o, 2.0, 0.0))
                               ).astype(jnp.float32)
                flag_vec = jnp.where(iota == bl, fb, flag_vec)
            flag_v[:] = flag_vec
            ocps = [
                pltpu.async_copy(sm_v, sm_hbm.at[pl.ds(core * 1024, 1024)],
                                 sem),
                pltpu.async_copy(id_v.at[pl.ds(0, 8)],
                                 id_hbm.at[pl.ds(core * 8, 8)], sem),
                pltpu.async_copy(flag_v.at[pl.ds(0, 8)],
                                 flag_hbm.at[pl.ds(core * 8, 8)], sem),
            ]
            for cp in ocps:
                cp.wait()


def kernel(s_e, adjacency_matrix, W1, b1, W2, b2):
    B, N, _, D = s_e.shape
    raw = pl.pallas_call(
        _score_kernel,
        grid=(1,),
        in_specs=[
            pl.BlockSpec((B, 2, N, D), lambda i: (0, 0, 0, 0)),
            pl.BlockSpec((D, D), lambda i: (0, 0)),
            pl.BlockSpec((1, D), lambda i: (0, 0)),
            pl.BlockSpec((D, 1), lambda i: (0, 0)),
            pl.BlockSpec((1, 1), lambda i: (0, 0)),
        ],
        out_specs=pl.BlockSpec((1, B * 2 * N), lambda i: (0, 0)),
        out_shape=jax.ShapeDtypeStruct((1, B * 2 * N), jnp.float32),
    )(s_e, W1, b1.reshape(1, D), W2, b2.reshape(1, 1))

    scores, ids, flag16 = _select_kernel(raw.reshape(B * 2 * N),
                                         adjacency_matrix)
    return (ids.reshape(B, 2), scores.reshape(B, 2, N),
            flag16.reshape(2, 8)[:, 0:4].reshape(B))


# speedup vs baseline: 1.0483x; 1.0483x over previous
import functools

import jax
import jax.numpy as jnp
from jax import lax
from jax.experimental import pallas as pl
from jax.experimental.pallas import tpu as pltpu
from jax.experimental.pallas import tpu_sc as plsc


def _score_kernel(x_ref, w1_ref, b1_ref, w2_ref, b2_ref, s_ref):
    x = x_ref[:].reshape(2048, 128)
    h = jnp.maximum(
        lax.dot_general(x, w1_ref[:], (((1,), (0,)), ((), ())),
                        preferred_element_type=jnp.float32) + b1_ref[:],
        0.0)
    s_ref[:] = lax.dot_general(w2_ref[:], h, (((0,), (1,)), ((), ())),
                               preferred_element_type=jnp.float32) + b2_ref[:]


_SEL_OUT = (
    jax.ShapeDtypeStruct((2048,), jnp.float32),
    jax.ShapeDtypeStruct((16,), jnp.int32),
    jax.ShapeDtypeStruct((8,), jnp.float32),
)


@functools.partial(
    pl.kernel,
    mesh=plsc.VectorSubcoreMesh(core_axis_name="c", subcore_axis_name="s"),
    out_type=_SEL_OUT,
    compiler_params=pltpu.CompilerParams(needs_layout_passes=False),
    scratch_types=[
        pltpu.VMEM((2048,), jnp.float32),
        pltpu.VMEM((16, 128), jnp.float32),
        pltpu.VMEM((2048,), jnp.float32),
        pltpu.VMEM((16,), jnp.int32),
        pltpu.VMEM((16,), jnp.float32),
        pltpu.SemaphoreType.DMA,
    ],
)
def _select_kernel(s_hbm, adj_hbm, sm_hbm, id_hbm, flag_hbm,
                   s_v, a_v, sm_v, id_v, flag_v, sem):
    cid = lax.axis_index("c")
    sid = lax.axis_index("s")

    @pl.when((cid == 0) & (sid == 0))
    def _():
        cps = [pltpu.async_copy(s_hbm, s_v, sem)]
        for b in range(8):
            cps.append(pltpu.async_copy(adj_hbm.at[b, pl.ds(0, 2)],
                                        a_v.at[pl.ds(2 * b, 2)], sem))
        for cp in cps:
            cp.wait()
        iota = lax.broadcasted_iota(jnp.int32, (16,), 0)
        ids = []
        for seg in range(16):
            row = seg % 2
            v = None
            gi = None
            for c in range(8):
                off = seg * 128 + c * 16
                sv = s_v[pl.ds(off, 16)]
                av = a_v[seg, pl.ds(c * 16, 16)]
                if c == 0:
                    av = jnp.where(iota == (1 - row), 0.0, av)
                sm = sv * av
                sm_v[pl.ds(off, 16)] = sm
                if c == 0:
                    v, gi = sm, iota
                else:
                    cond = sm > v
                    gi = jnp.where(cond, c * 16 + iota, gi)
                    v = jnp.where(cond, sm, v)
            mx = jnp.max(v)
            ids.append(jnp.min(jnp.where(v == mx, gi, 2048)))
        ids_vec = jnp.zeros((16,), jnp.int32)
        for seg in range(16):
            ids_vec = jnp.where(iota == seg, ids[seg], ids_vec)
        id_v[:] = ids_vec
        flag_vec = jnp.zeros((16,), jnp.float32)
        for b in range(8):
            a = ids[2 * b] > 0
            o = ids[2 * b + 1] > 0
            fb = jnp.where(a & o, 3.0,
                           jnp.where(a, 1.0, jnp.where(o, 2.0, 0.0))
                           ).astype(jnp.float32)
            flag_vec = jnp.where(iota == b, fb, flag_vec)
        flag_v[:] = flag_vec
        ocps = [pltpu.async_copy(sm_v, sm_hbm, sem),
                pltpu.async_copy(id_v, id_hbm, sem),
                pltpu.async_copy(flag_v.at[pl.ds(0, 8)], flag_hbm, sem)]
        for cp in ocps:
            cp.wait()


def kernel(s_e, adjacency_matrix, W1, b1, W2, b2):
    B, N, _, D = s_e.shape
    raw = pl.pallas_call(
        _score_kernel,
        grid=(1,),
        in_specs=[
            pl.BlockSpec((B, 2, N, D), lambda i: (0, 0, 0, 0)),
            pl.BlockSpec((D, D), lambda i: (0, 0)),
            pl.BlockSpec((1, D), lambda i: (0, 0)),
            pl.BlockSpec((D, 1), lambda i: (0, 0)),
            pl.BlockSpec((1, 1), lambda i: (0, 0)),
        ],
        out_specs=pl.BlockSpec((1, B * 2 * N), lambda i: (0, 0)),
        out_shape=jax.ShapeDtypeStruct((1, B * 2 * N), jnp.float32),
    )(s_e, W1, b1.reshape(1, D), W2, b2.reshape(1, 1))

    scores, ids, flag = _select_kernel(raw.reshape(B * 2 * N),
                                       adjacency_matrix)
    return ids.reshape(B, 2), scores.reshape(B, 2, N), flag
